# baseline (speedup 1.0000x reference)
import functools

import jax
import jax.numpy as jnp
from jax.experimental import pallas as pl
from jax.experimental.pallas import tpu as pltpu

N = 10000
E = 160000
D = 256
THETAS = [[0.9375, -1.40625, 0.703125, -0.1171875],
          [0.234375, 0.46875, -0.9375, 0.3515625],
          [0.09375, 0.375, 0.0, -0.3515625],
          [0.015625, 0.09375, 0.234375, 0.1171875]]
K_SPLIT = 1

_BN = 1000


def _lin_body(h_ref, w_ref, b_ref, o_ref):
    acc = jnp.dot(h_ref[...], w_ref[...], preferred_element_type=jnp.float32)
    acc = acc + b_ref[...]
    o_ref[...] = jnp.where(acc >= 0, acc, 0.01 * acc)


def _leaky_linear(h, W, b):
    K = h.shape[1]
    Wt = W.T
    return pl.pallas_call(
        _lin_body,
        grid=(N // _BN,),
        in_specs=[
            pl.BlockSpec((_BN, K), lambda i: (i, 0)),
            pl.BlockSpec((K, D), lambda i: (0, 0)),
            pl.BlockSpec((D,), lambda i: (0,)),
        ],
        out_specs=pl.BlockSpec((_BN, D), lambda i: (i, 0)),
        out_shape=jax.ShapeDtypeStruct((N, D), jnp.float32),
    )(h, Wt, b)


def _transh_body(f_ref, w_ref, b_ref, o_ref):
    o_ref[...] = jnp.dot(f_ref[...], w_ref[...],
                         preferred_element_type=jnp.float32) + b_ref[...]


def _transh(feat, W_t, b_t):
    return pl.pallas_call(
        _transh_body,
        grid=(N // _BN,),
        in_specs=[
            pl.BlockSpec((_BN, D), lambda i: (i, 0)),
            pl.BlockSpec((D, D), lambda i: (0, 0)),
            pl.BlockSpec((D,), lambda i: (0,)),
        ],
        out_specs=pl.BlockSpec((_BN, D), lambda i: (i, 0)),
        out_shape=jax.ShapeDtypeStruct((N, D), jnp.float32),
    )(feat, W_t.T, b_t)


def kernel(feat, edge_index, w_r_src, w_r_dst, W_lin, b_lin, W_lin1, b_lin1,
           W_t, b_t):
    src = edge_index[0]
    dst = edge_index[1]
    feat0 = feat
    score = feat0 @ w_r_src
    score = score[src] + (feat0 @ w_r_dst)[dst]
    sign = jnp.sign(score)
    pos = (sign >= 0).astype(jnp.float32)
    neg = (sign < 0).astype(jnp.float32)
    in_deg = jax.ops.segment_sum(jnp.ones((E,), jnp.float32), dst, num_segments=N)
    pos_deg = jax.ops.segment_sum(pos, dst, num_segments=N)
    neg_deg = jax.ops.segment_sum(neg, dst, num_segments=N)
    Di = jnp.power(jnp.clip(in_deg, 1.0), -0.5)[:, None]
    Dip = jnp.power(jnp.clip(pos_deg, 1.0), -0.5)[:, None]
    Din = jnp.power(jnp.clip(neg_deg, 1.0), -0.5)[:, None]

    def lap(f, Dx, mask):
        m = (f * Dx)[src]
        if mask is not None:
            m = m * mask[:, None]
        return f - jax.ops.segment_sum(m, dst, num_segments=N) * Dx

    hs_o = []
    f = feat0
    for theta in THETAS:
        h = theta[0] * f
        for k in range(1, 4):
            f = lap(f, Di, None)
            h = h + theta[k] * f
        hs_o.append(h)
    hs_p = []
    f = feat0
    for theta in THETAS[:K_SPLIT + 1]:
        h = theta[0] * f
        for k in range(1, 4):
            f = lap(f, Dip, pos)
            h = h + theta[k] * f
        hs_p.append(h)
    hs_n = []
    f = feat0
    for theta in THETAS[K_SPLIT + 1:]:
        h = theta[0] * f
        for k in range(1, 4):
            f = lap(f, Din, neg)
            h = h + theta[k] * f
        hs_n.append(h)

    hs_o_cat = jnp.concatenate(hs_o, axis=1)
    hs_pn = jnp.concatenate(hs_p + hs_n, axis=1)
    hs_o_out = _leaky_linear(hs_o_cat, W_lin, b_lin)
    hs_pn_out = _leaky_linear(hs_pn, W_lin1, b_lin1)
    transh = _transh(feat0, W_t, b_t)
    return (hs_o_out, hs_pn_out, transh)
